# baseline (speedup 1.0000x reference)
import functools

import jax
import jax.numpy as jnp
from jax import lax
from jax.experimental import pallas as pl
from jax.experimental.pallas import tpu as pltpu
from jax.experimental.pallas import tpu_sc as plsc

_NC = 2
_NS = 16
_NW = _NC * _NS
_L = 16

_B = 1024
_H = 200
_D = 64
_V = 1000000
_ROWS = _B * _H
_RPW = _ROWS // _NW
_CHUNK = 128
_NCHUNK = _RPW // _CHUNK
_NBUF = 5
_NGRP = _NCHUNK // _NBUF
_VB = 1024


def _tc_reformat(tT_ref, out_ref):
    x = tT_ref[...]
    y = jnp.swapaxes(x, 0, 1)
    out_ref[...] = jnp.concatenate([y, y], axis=1)


def _reformat_table(tabT):
    grid = (_V + _VB - 1) // _VB
    return pl.pallas_call(
        _tc_reformat,
        grid=(grid,),
        in_specs=[pl.BlockSpec((_D, _VB), lambda i: (0, i))],
        out_specs=pl.BlockSpec((_VB, 2 * _D), lambda i: (i, 0)),
        out_shape=jax.ShapeDtypeStruct((_V, 2 * _D), jnp.float32),
    )(tabT)


def _make_sc_kernel():
    mesh = plsc.VectorSubcoreMesh(core_axis_name="c", subcore_axis_name="s")

    @functools.partial(
        pl.kernel,
        out_type=jax.ShapeDtypeStruct((_ROWS, _D), jnp.float32),
        mesh=mesh,
        scratch_types=[
            pltpu.VMEM((_RPW,), jnp.int32),
            [pltpu.VMEM((_CHUNK, 2 * _D), jnp.float32) for _ in range(_NBUF)],
            pltpu.VMEM((_D,), jnp.float32),
            pltpu.SemaphoreType.DMA((_NBUF,)),
            pltpu.SemaphoreType.DMA((_NBUF,)),
        ],
        compiler_params=pltpu.CompilerParams(use_tc_tiling_on_sc=False),
    )
    def emb_kernel(x_hbm, table_hbm, pe_hbm, out_hbm,
                   idx_v, bufs, pe_v, gsem, ssem):
        wid = lax.axis_index("s") * _NC + lax.axis_index("c")
        base = wid * _RPW

        pltpu.sync_copy(x_hbm.at[pl.ds(base, _RPW)], idx_v)
        pltpu.sync_copy(pe_hbm, pe_v)
        pe_vecs = [pe_v[pl.ds(k * _L, _L)] for k in range(_D // _L)]

        def gather_start(j, b):
            pltpu.async_copy(
                table_hbm.at[idx_v.at[pl.ds(j * _CHUNK, _CHUNK)]],
                bufs[b], gsem.at[b])

        def gather_wait(b):
            pltpu.make_async_copy(
                table_hbm.at[idx_v.at[pl.ds(0, _CHUNK)]],
                bufs[b], gsem.at[b]).wait()

        def store_start(j, b):
            pltpu.async_copy(
                bufs[b].at[:, pl.ds(0, _D)],
                out_hbm.at[pl.ds(base + j * _CHUNK, _CHUNK)],
                ssem.at[b])

        def store_wait(b):
            pltpu.make_async_copy(
                bufs[b].at[:, pl.ds(0, _D)],
                out_hbm.at[pl.ds(base, _CHUNK)], ssem.at[b]).wait()

        for b in range(_NBUF):
            gather_start(b, b)

        @pl.loop(0, _NGRP)
        def _grp(g):
            jbase = g * _NBUF
            for b in range(_NBUF):
                gather_wait(b)
                buf = bufs[b]

                @plsc.parallel_loop(0, _CHUNK, 1, unroll=4)
                def _rows(r):
                    for k in range(_D // _L):
                        sl = pl.ds(k * _L, _L)
                        buf[r, sl] = buf[r, sl] + pe_vecs[k]

                store_start(jbase + b, b)

                @pl.when(g < _NGRP - 1)
                def _next():
                    store_wait(b)
                    gather_start(jbase + _NBUF + b, b)

        for b in range(_NBUF):
            store_wait(b)

    return emb_kernel


_emb_kernel = _make_sc_kernel()


def kernel(x, table, pe):
    x_flat = x.reshape(_ROWS)
    tab_lin = _reformat_table(table.T)
    pe_row = pe[x.shape[0]]
    out = _emb_kernel(x_flat, tab_lin, pe_row)
    return out.reshape(_B, _H, _D)


# speedup vs baseline: 1.4074x; 1.4074x over previous
import functools

import jax
import jax.numpy as jnp
from jax import lax
from jax.experimental import pallas as pl
from jax.experimental.pallas import tpu as pltpu
from jax.experimental.pallas import tpu_sc as plsc

_NC = 2
_NS = 16
_NW = _NC * _NS
_L = 16

_B = 1024
_H = 200
_D = 64
_V = 1000000
_ROWS = _B * _H
_RPW = _ROWS // _NW
_CHUNK = 128
_NCHUNK = _RPW // _CHUNK
_NBUF = 5
_NGRP = _NCHUNK // _NBUF
_VB = 2048


import numpy as _np

_DUP_EYE = _np.concatenate([_np.eye(_D, dtype=_np.float32)] * 2, axis=1)


def _tc_reformat(eye_ref, tT_ref, out_ref):
    x = tT_ref[...]
    out_ref[...] = lax.dot_general(
        x, eye_ref[...], (((0,), (0,)), ((), ())),
        preferred_element_type=jnp.float32)


def _reformat_table(tabT):
    grid = (_V + _VB - 1) // _VB
    return pl.pallas_call(
        _tc_reformat,
        grid=(grid,),
        in_specs=[pl.BlockSpec((_D, 2 * _D), lambda i: (0, 0)),
                  pl.BlockSpec((_D, _VB), lambda i: (0, i))],
        out_specs=pl.BlockSpec((_VB, 2 * _D), lambda i: (i, 0)),
        out_shape=jax.ShapeDtypeStruct((_V, 2 * _D), jnp.float32),
    )(jnp.asarray(_DUP_EYE), tabT)


def _make_sc_kernel():
    mesh = plsc.VectorSubcoreMesh(core_axis_name="c", subcore_axis_name="s")

    @functools.partial(
        pl.kernel,
        out_type=jax.ShapeDtypeStruct((_ROWS, _D), jnp.float32),
        mesh=mesh,
        scratch_types=[
            pltpu.VMEM((_RPW,), jnp.int32),
            [pltpu.VMEM((_CHUNK, 2 * _D), jnp.float32) for _ in range(_NBUF)],
            pltpu.VMEM((_D,), jnp.float32),
            pltpu.SemaphoreType.DMA((_NBUF,)),
            pltpu.SemaphoreType.DMA((_NBUF,)),
        ],
        compiler_params=pltpu.CompilerParams(use_tc_tiling_on_sc=False),
    )
    def emb_kernel(x_hbm, table_hbm, pe_hbm, out_hbm,
                   idx_v, bufs, pe_v, gsem, ssem):
        wid = lax.axis_index("s") * _NC + lax.axis_index("c")
        base = wid * _RPW

        pltpu.sync_copy(x_hbm.at[pl.ds(base, _RPW)], idx_v)
        pltpu.sync_copy(pe_hbm, pe_v)
        pe_vecs = [pe_v[pl.ds(k * _L, _L)] for k in range(_D // _L)]

        def gather_start(j, b):
            pltpu.async_copy(
                table_hbm.at[idx_v.at[pl.ds(j * _CHUNK, _CHUNK)]],
                bufs[b], gsem.at[b])

        def gather_wait(b):
            pltpu.make_async_copy(
                table_hbm.at[idx_v.at[pl.ds(0, _CHUNK)]],
                bufs[b], gsem.at[b]).wait()

        def store_start(j, b):
            pltpu.async_copy(
                bufs[b].at[:, pl.ds(0, _D)],
                out_hbm.at[pl.ds(base + j * _CHUNK, _CHUNK)],
                ssem.at[b])

        def store_wait(b):
            pltpu.make_async_copy(
                bufs[b].at[:, pl.ds(0, _D)],
                out_hbm.at[pl.ds(base, _CHUNK)], ssem.at[b]).wait()

        for b in range(_NBUF):
            gather_start(b, b)

        @pl.loop(0, _NGRP)
        def _grp(g):
            jbase = g * _NBUF
            for b in range(_NBUF):
                gather_wait(b)
                buf = bufs[b]

                @plsc.parallel_loop(0, _CHUNK, 1, unroll=4)
                def _rows(r):
                    for k in range(_D // _L):
                        sl = pl.ds(k * _L, _L)
                        buf[r, sl] = buf[r, sl] + pe_vecs[k]

                store_start(jbase + b, b)

                @pl.when(g < _NGRP - 1)
                def _next():
                    store_wait(b)
                    gather_start(jbase + _NBUF + b, b)

        for b in range(_NBUF):
            store_wait(b)

    return emb_kernel


_emb_kernel = _make_sc_kernel()


def kernel(x, table, pe):
    x_flat = x.reshape(_ROWS)
    tab_lin = _reformat_table(table.T)
    pe_row = pe[x.shape[0]]
    out = _emb_kernel(x_flat, tab_lin, pe_row)
    return out.reshape(_B, _H, _D)


# speedup vs baseline: 2.0453x; 1.4533x over previous
import functools

import jax
import jax.numpy as jnp
from jax import lax
from jax.experimental import pallas as pl
from jax.experimental.pallas import tpu as pltpu
from jax.experimental.pallas import tpu_sc as plsc

_NC = 2
_NS = 16
_NW = _NC * _NS
_L = 16

_B = 1024
_H = 200
_D = 64
_V = 1000000
_ROWS = _B * _H
_RPW = _ROWS // _NW
_CHUNK = 128
_NCHUNK = _RPW // _CHUNK
_NBUF = 5
_NGRP = _NCHUNK // _NBUF
_VB = 8192


import numpy as _np

_DUP_EYE = _np.concatenate([_np.eye(_D, dtype=_np.float32)] * 2, axis=1)


def _tc_reformat(eye_ref, tT_ref, out_ref):
    x = tT_ref[...]
    out_ref[...] = lax.dot_general(
        x, eye_ref[...], (((0,), (0,)), ((), ())),
        preferred_element_type=jnp.float32)


def _reformat_table(tabT):
    grid = (_V + _VB - 1) // _VB
    return pl.pallas_call(
        _tc_reformat,
        grid=(grid,),
        in_specs=[pl.BlockSpec((_D, 2 * _D), lambda i: (0, 0)),
                  pl.BlockSpec((_D, _VB), lambda i: (0, i))],
        out_specs=pl.BlockSpec((_VB, 2 * _D), lambda i: (i, 0)),
        out_shape=jax.ShapeDtypeStruct((_V, 2 * _D), jnp.float32),
    )(jnp.asarray(_DUP_EYE), tabT)


def _make_sc_kernel():
    mesh = plsc.VectorSubcoreMesh(core_axis_name="c", subcore_axis_name="s")

    @functools.partial(
        pl.kernel,
        out_type=jax.ShapeDtypeStruct((_ROWS, _D), jnp.float32),
        mesh=mesh,
        scratch_types=[
            pltpu.VMEM((_RPW,), jnp.int32),
            [pltpu.VMEM((_CHUNK, 2 * _D), jnp.float32) for _ in range(_NBUF)],
            pltpu.VMEM((_D,), jnp.float32),
            pltpu.SemaphoreType.DMA((_NBUF,)),
            pltpu.SemaphoreType.DMA((_NBUF,)),
        ],
        compiler_params=pltpu.CompilerParams(use_tc_tiling_on_sc=False),
    )
    def emb_kernel(x_hbm, table_hbm, pe_hbm, out_hbm,
                   idx_v, bufs, pe_v, gsem, ssem):
        wid = lax.axis_index("s") * _NC + lax.axis_index("c")
        base = wid * _RPW

        pltpu.sync_copy(x_hbm.at[pl.ds(base, _RPW)], idx_v)
        pltpu.sync_copy(pe_hbm, pe_v)
        pe_vecs = [pe_v[pl.ds(k * _L, _L)] for k in range(_D // _L)]

        def gather_start(j, b):
            pltpu.async_copy(
                table_hbm.at[idx_v.at[pl.ds(j * _CHUNK, _CHUNK)]],
                bufs[b], gsem.at[b])

        def gather_wait(b):
            pltpu.make_async_copy(
                table_hbm.at[idx_v.at[pl.ds(0, _CHUNK)]],
                bufs[b], gsem.at[b]).wait()

        def store_start(j, b):
            pltpu.async_copy(
                bufs[b].at[:, pl.ds(0, _D)],
                out_hbm.at[pl.ds(base + j * _CHUNK, _CHUNK)],
                ssem.at[b])

        def store_wait(b):
            pltpu.make_async_copy(
                bufs[b].at[:, pl.ds(0, _D)],
                out_hbm.at[pl.ds(base, _CHUNK)], ssem.at[b]).wait()

        for b in range(_NBUF):
            gather_start(b, b)

        @pl.loop(0, _NGRP)
        def _grp(g):
            jbase = g * _NBUF
            for b in range(_NBUF):
                gather_wait(b)
                buf = bufs[b]

                @plsc.parallel_loop(0, _CHUNK, 1, unroll=4)
                def _rows(r):
                    for k in range(_D // _L):
                        sl = pl.ds(k * _L, _L)
                        buf[r, sl] = buf[r, sl] + pe_vecs[k]

                store_start(jbase + b, b)

                @pl.when(g < _NGRP - 1)
                def _next():
                    store_wait(b)
                    gather_start(jbase + _NBUF + b, b)

        for b in range(_NBUF):
            store_wait(b)

    return emb_kernel


_emb_kernel = _make_sc_kernel()


def kernel(x, table, pe):
    x_flat = x.reshape(_ROWS)
    tab_lin = _reformat_table(table.T)
    pe_row = pe[x.shape[0]]
    out = _emb_kernel(x_flat, tab_lin, pe_row)
    return out.reshape(_B, _H, _D)


# speedup vs baseline: 2.1562x; 1.0542x over previous
import functools

import jax
import jax.numpy as jnp
from jax import lax
from jax.experimental import pallas as pl
from jax.experimental.pallas import tpu as pltpu
from jax.experimental.pallas import tpu_sc as plsc

_NC = 2
_NS = 16
_NW = _NC * _NS
_L = 16

_B = 1024
_H = 200
_D = 64
_V = 1000000
_ROWS = _B * _H
_RPW = _ROWS // _NW
_CHUNK = 128
_NCHUNK = _RPW // _CHUNK
_NBUF = 5
_NGRP = _NCHUNK // _NBUF
_VB = 16384


import numpy as _np

_DUP_EYE = _np.concatenate([_np.eye(_D, dtype=_np.float32)] * 2, axis=1)


def _tc_reformat(eye_ref, tT_ref, out_ref):
    x = tT_ref[...]
    out_ref[...] = lax.dot_general(
        x, eye_ref[...], (((0,), (0,)), ((), ())),
        preferred_element_type=jnp.float32)


def _reformat_table(tabT):
    grid = (_V + _VB - 1) // _VB
    return pl.pallas_call(
        _tc_reformat,
        grid=(grid,),
        in_specs=[pl.BlockSpec((_D, 2 * _D), lambda i: (0, 0)),
                  pl.BlockSpec((_D, _VB), lambda i: (0, i))],
        out_specs=pl.BlockSpec((_VB, 2 * _D), lambda i: (i, 0)),
        out_shape=jax.ShapeDtypeStruct((_V, 2 * _D), jnp.float32),
    )(jnp.asarray(_DUP_EYE), tabT)


def _make_sc_kernel():
    mesh = plsc.VectorSubcoreMesh(core_axis_name="c", subcore_axis_name="s")

    @functools.partial(
        pl.kernel,
        out_type=jax.ShapeDtypeStruct((_ROWS, _D), jnp.float32),
        mesh=mesh,
        scratch_types=[
            pltpu.VMEM((_RPW,), jnp.int32),
            [pltpu.VMEM((_CHUNK, 2 * _D), jnp.float32) for _ in range(_NBUF)],
            pltpu.VMEM((_D,), jnp.float32),
            pltpu.SemaphoreType.DMA((_NBUF,)),
            pltpu.SemaphoreType.DMA((_NBUF,)),
        ],
        compiler_params=pltpu.CompilerParams(use_tc_tiling_on_sc=False),
    )
    def emb_kernel(x_hbm, table_hbm, pe_hbm, out_hbm,
                   idx_v, bufs, pe_v, gsem, ssem):
        wid = lax.axis_index("s") * _NC + lax.axis_index("c")
        base = wid * _RPW

        pltpu.sync_copy(x_hbm.at[pl.ds(base, _RPW)], idx_v)
        pltpu.sync_copy(pe_hbm, pe_v)
        pe_vecs = [pe_v[pl.ds(k * _L, _L)] for k in range(_D // _L)]

        def gather_start(j, b):
            pltpu.async_copy(
                table_hbm.at[idx_v.at[pl.ds(j * _CHUNK, _CHUNK)]],
                bufs[b], gsem.at[b])

        def gather_wait(b):
            pltpu.make_async_copy(
                table_hbm.at[idx_v.at[pl.ds(0, _CHUNK)]],
                bufs[b], gsem.at[b]).wait()

        def store_start(j, b):
            pltpu.async_copy(
                bufs[b].at[:, pl.ds(0, _D)],
                out_hbm.at[pl.ds(base + j * _CHUNK, _CHUNK)],
                ssem.at[b])

        def store_wait(b):
            pltpu.make_async_copy(
                bufs[b].at[:, pl.ds(0, _D)],
                out_hbm.at[pl.ds(base, _CHUNK)], ssem.at[b]).wait()

        for b in range(_NBUF):
            gather_start(b, b)

        @pl.loop(0, _NGRP)
        def _grp(g):
            jbase = g * _NBUF
            for b in range(_NBUF):
                gather_wait(b)
                buf = bufs[b]

                @plsc.parallel_loop(0, _CHUNK, 1, unroll=4)
                def _rows(r):
                    for k in range(_D // _L):
                        sl = pl.ds(k * _L, _L)
                        buf[r, sl] = buf[r, sl] + pe_vecs[k]

                store_start(jbase + b, b)

                @pl.when(g < _NGRP - 1)
                def _next():
                    store_wait(b)
                    gather_start(jbase + _NBUF + b, b)

        for b in range(_NBUF):
            store_wait(b)

    return emb_kernel


_emb_kernel = _make_sc_kernel()


def kernel(x, table, pe):
    x_flat = x.reshape(_ROWS)
    tab_lin = _reformat_table(table.T)
    pe_row = pe[x.shape[0]]
    out = _emb_kernel(x_flat, tab_lin, pe_row)
    return out.reshape(_B, _H, _D)


# speedup vs baseline: 2.1815x; 1.0117x over previous
import functools

import jax
import jax.numpy as jnp
from jax import lax
from jax.experimental import pallas as pl
from jax.experimental.pallas import tpu as pltpu
from jax.experimental.pallas import tpu_sc as plsc

_NC = 2
_NS = 16
_NW = _NC * _NS
_L = 16

_B = 1024
_H = 200
_D = 64
_V = 1000000
_ROWS = _B * _H
_RPW = _ROWS // _NW
_CHUNK = 128
_NCHUNK = _RPW // _CHUNK
_NBUF = 5
_NGRP = _NCHUNK // _NBUF
_VB = 32768


import numpy as _np

_DUP_EYE = _np.concatenate([_np.eye(_D, dtype=_np.float32)] * 2, axis=1)


def _tc_reformat(eye_ref, tT_ref, out_ref):
    x = tT_ref[...]
    out_ref[...] = lax.dot_general(
        x, eye_ref[...], (((0,), (0,)), ((), ())),
        preferred_element_type=jnp.float32)


def _reformat_table(tabT):
    grid = (_V + _VB - 1) // _VB
    return pl.pallas_call(
        _tc_reformat,
        grid=(grid,),
        in_specs=[pl.BlockSpec((_D, 2 * _D), lambda i: (0, 0)),
                  pl.BlockSpec((_D, _VB), lambda i: (0, i))],
        out_specs=pl.BlockSpec((_VB, 2 * _D), lambda i: (i, 0)),
        out_shape=jax.ShapeDtypeStruct((_V, 2 * _D), jnp.float32),
    )(jnp.asarray(_DUP_EYE), tabT)


def _make_sc_kernel():
    mesh = plsc.VectorSubcoreMesh(core_axis_name="c", subcore_axis_name="s")

    @functools.partial(
        pl.kernel,
        out_type=jax.ShapeDtypeStruct((_ROWS, _D), jnp.float32),
        mesh=mesh,
        scratch_types=[
            pltpu.VMEM((_RPW,), jnp.int32),
            [pltpu.VMEM((_CHUNK, 2 * _D), jnp.float32) for _ in range(_NBUF)],
            pltpu.VMEM((_D,), jnp.float32),
            pltpu.SemaphoreType.DMA((_NBUF,)),
            pltpu.SemaphoreType.DMA((_NBUF,)),
        ],
        compiler_params=pltpu.CompilerParams(use_tc_tiling_on_sc=False),
    )
    def emb_kernel(x_hbm, table_hbm, pe_hbm, out_hbm,
                   idx_v, bufs, pe_v, gsem, ssem):
        wid = lax.axis_index("s") * _NC + lax.axis_index("c")
        base = wid * _RPW

        pltpu.sync_copy(x_hbm.at[pl.ds(base, _RPW)], idx_v)
        pltpu.sync_copy(pe_hbm, pe_v)
        pe_vecs = [pe_v[pl.ds(k * _L, _L)] for k in range(_D // _L)]

        def gather_start(j, b):
            pltpu.async_copy(
                table_hbm.at[idx_v.at[pl.ds(j * _CHUNK, _CHUNK)]],
                bufs[b], gsem.at[b])

        def gather_wait(b):
            pltpu.make_async_copy(
                table_hbm.at[idx_v.at[pl.ds(0, _CHUNK)]],
                bufs[b], gsem.at[b]).wait()

        def store_start(j, b):
            pltpu.async_copy(
                bufs[b].at[:, pl.ds(0, _D)],
                out_hbm.at[pl.ds(base + j * _CHUNK, _CHUNK)],
                ssem.at[b])

        def store_wait(b):
            pltpu.make_async_copy(
                bufs[b].at[:, pl.ds(0, _D)],
                out_hbm.at[pl.ds(base, _CHUNK)], ssem.at[b]).wait()

        for b in range(_NBUF):
            gather_start(b, b)

        @pl.loop(0, _NGRP)
        def _grp(g):
            jbase = g * _NBUF
            for b in range(_NBUF):
                gather_wait(b)
                buf = bufs[b]

                @plsc.parallel_loop(0, _CHUNK, 1, unroll=4)
                def _rows(r):
                    for k in range(_D // _L):
                        sl = pl.ds(k * _L, _L)
                        buf[r, sl] = buf[r, sl] + pe_vecs[k]

                store_start(jbase + b, b)

                @pl.when(g < _NGRP - 1)
                def _next():
                    store_wait(b)
                    gather_start(jbase + _NBUF + b, b)

        for b in range(_NBUF):
            store_wait(b)

    return emb_kernel


_emb_kernel = _make_sc_kernel()


def kernel(x, table, pe):
    x_flat = x.reshape(_ROWS)
    tab_lin = _reformat_table(table.T)
    pe_row = pe[x.shape[0]]
    out = _emb_kernel(x_flat, tab_lin, pe_row)
    return out.reshape(_B, _H, _D)
